# baseline (speedup 1.0000x reference)
import functools

import jax
import jax.numpy as jnp
from jax import lax
from jax.experimental import pallas as pl
from jax.experimental.pallas import tpu as pltpu
from jax.experimental.pallas import tpu_sc as plsc

B = 4096
T = 40960
D = 32
L = 20
NC = 2
NS = 16
NW = NC * NS
BPW = B // NW
NE = BPW * L
NCHUNK = NE // 16


def _sc_body(idx0, off0, idx1, off1, tab0, tab1, out,
             offs_v, pos_v, msk_v, idxs_v, rows_v, sem):
    wid = lax.axis_index("s") * NC + lax.axis_index("c")
    base = wid * BPW
    lane = lax.iota(jnp.int32, 16)

    for f, (idx_hbm, off_hbm, tab_hbm) in enumerate(
            ((idx0, off0, tab0), (idx1, off1, tab1))):
        pltpu.sync_copy(off_hbm, offs_v.at[pl.ds(0, B + 1)])

        def pos_body(i, carry):
            b, t = carry
            st = plsc.load_gather(offs_v, [base + b])
            en = plsc.load_gather(offs_v, [base + b + 1])
            pos = st + t
            valid = pos < en
            posc = jnp.minimum(pos, T - 1)
            plsc.store_scatter(pos_v, [b, t], posc)
            msk_v[pl.ds(i * 16, 16)] = jnp.where(valid, 1.0, 0.0)
            t2 = t + 16
            over = t2 >= L
            t_new = jnp.where(over, t2 - L, t2)
            b_new = b + jnp.where(over, 1, 0)
            return (b_new, t_new)

        lax.fori_loop(0, NCHUNK, pos_body,
                      (jnp.zeros((16,), jnp.int32), lane))

        def fire_idx(r, carry):
            pltpu.make_async_copy(idx_hbm.at[pos_v.at[r]], idxs_v.at[r],
                                  sem).start()
            return carry

        lax.fori_loop(0, BPW, fire_idx, 0)

        def drain_idx(r, carry):
            pltpu.make_async_copy(idx_hbm.at[pos_v.at[r]], idxs_v.at[r],
                                  sem).wait()
            return carry

        lax.fori_loop(0, BPW, drain_idx, 0)

        def fire_rows(r, carry):
            pltpu.make_async_copy(tab_hbm.at[idxs_v.at[r]],
                                  rows_v.at[pl.ds(r * L, L)], sem).start()
            return carry

        lax.fori_loop(0, BPW, fire_rows, 0)

        def drain_rows(r, carry):
            pltpu.make_async_copy(tab_hbm.at[idxs_v.at[r]],
                                  rows_v.at[pl.ds(r * L, L)], sem).wait()
            return carry

        lax.fori_loop(0, BPW, drain_rows, 0)

        def msk_body(i, carry):
            e = i * 16 + lane
            m = msk_v[pl.ds(i * 16, 16)]
            for c in range(D):
                cc = jnp.full((16,), c, jnp.int32)
                v = plsc.load_gather(rows_v, [e, cc])
                plsc.store_scatter(rows_v, [e, cc], v * m)
            return carry

        lax.fori_loop(0, NCHUNK, msk_body, 0)

        def fire_out(r, carry):
            pltpu.make_async_copy(rows_v.at[pl.ds(r * L, L)],
                                  out.at[base + r, f], sem).start()
            return carry

        lax.fori_loop(0, BPW, fire_out, 0)

        def drain_out(r, carry):
            pltpu.make_async_copy(rows_v.at[pl.ds(r * L, L)],
                                  out.at[base + r, f], sem).wait()
            return carry

        lax.fori_loop(0, BPW, drain_out, 0)


_sc_call = functools.partial(
    pl.kernel,
    mesh=plsc.VectorSubcoreMesh(core_axis_name="c", subcore_axis_name="s"),
    compiler_params=pltpu.CompilerParams(needs_layout_passes=False,
                                         use_tc_tiling_on_sc=False),
    out_type=jax.ShapeDtypeStruct((B, 2, L, D), jnp.float32),
    scratch_types=[
        pltpu.VMEM((B + 128,), jnp.int32),
        pltpu.VMEM((BPW, L), jnp.int32),
        pltpu.VMEM((NE,), jnp.float32),
        pltpu.VMEM((BPW, L), jnp.int32),
        pltpu.VMEM((NE, D), jnp.float32),
        pltpu.SemaphoreType.DMA,
    ],
)(_sc_body)


def kernel(indices_0, offsets_0, indices_1, offsets_1, table_0, table_1):
    out = _sc_call(indices_0, offsets_0, indices_1, offsets_1,
                   table_0, table_1)
    return out.reshape(B, 2 * L * D)


# speedup vs baseline: 1.0672x; 1.0672x over previous
import functools

import jax
import jax.numpy as jnp
from jax import lax
from jax.experimental import pallas as pl
from jax.experimental.pallas import tpu as pltpu
from jax.experimental.pallas import tpu_sc as plsc

B = 4096
T = 40960
V = 1000000
D = 32
L = 20
NC = 2
NS = 16
NW = NC * NS
BPW = B // NW
NE = BPW * L
NCHUNK = NE // 16
RPS = 8
EPS = RPS * L
NSUB = BPW // RPS


def _sc_body(idx0, off0, idx1, off1, tab0, tab1, out,
             offs_v, pos_v, msk_v, rid_v, col_v, big_v, asm_v, sem):
    wid = lax.axis_index("s") * NC + lax.axis_index("c")
    base = wid * BPW
    lane = lax.iota(jnp.int32, 16)

    for f, (idx_hbm, off_hbm, tab_hbm) in enumerate(
            ((idx0, off0, tab0), (idx1, off1, tab1))):
        pltpu.sync_copy(off_hbm, offs_v.at[pl.ds(0, B + 1)])

        def pos_body(i, carry):
            b, t = carry
            st = plsc.load_gather(offs_v, [base + b])
            en = plsc.load_gather(offs_v, [base + b + 1])
            pos = st + t
            valid = pos < en
            posc = jnp.minimum(pos, T - 1)
            pos_v[pl.ds(i * 16, 16)] = posc
            msk_v[pl.ds(i * 16, 16)] = jnp.where(valid, 1.0, 0.0)
            t2 = t + 16
            over = t2 >= L
            t_new = jnp.where(over, t2 - L, t2)
            b_new = b + jnp.where(over, 1, 0)
            return (b_new, t_new)

        lax.fori_loop(0, NCHUNK, pos_body,
                      (jnp.zeros((16,), jnp.int32), lane))

        def fire_idx(j, carry):
            pltpu.make_async_copy(idx_hbm.at[pos_v.at[pl.ds(j * 128, 128)]],
                                  rid_v.at[pl.ds(j * 128, 128)], sem).start()
            return carry

        lax.fori_loop(0, NE // 128, fire_idx, 0)

        def drain_idx(j, carry):
            pltpu.make_async_copy(idx_hbm.at[pos_v.at[pl.ds(j * 128, 128)]],
                                  rid_v.at[pl.ds(j * 128, 128)], sem).wait()
            return carry

        lax.fori_loop(0, NE // 128, drain_idx, 0)

        def rid_body(i, carry):
            v = rid_v[pl.ds(i * 16, 16)]
            rid_v[pl.ds(i * 16, 16)] = lax.shift_right_logical(v, 2)
            col_v[pl.ds(i * 16, 16)] = lax.shift_left(v & 3, 5)
            return carry

        lax.fori_loop(0, NCHUNK, rid_body, 0)

        def sub_body(k, carry):
            e0 = k * EPS
            c1 = pltpu.make_async_copy(
                tab_hbm.at[rid_v.at[pl.ds(e0, 128)]],
                big_v.at[pl.ds(0, 128)], sem)
            c2 = pltpu.make_async_copy(
                tab_hbm.at[rid_v.at[pl.ds(e0 + 128, 32)]],
                big_v.at[pl.ds(128, 32)], sem)
            c1.start()
            c2.start()
            c1.wait()
            c2.wait()

            def ext_body(i, carry2):
                s, t = carry2
                ii = i * 16 + lane
                m = msk_v[pl.ds(e0 + i * 16, 16)]
                cb = col_v[pl.ds(e0 + i * 16, 16)]
                cvec = lax.shift_right_logical(t, 2)
                lbase = lax.shift_left(t & 3, 5)
                for d in range(D):
                    val = plsc.load_gather(big_v, [ii, cb + d])
                    plsc.store_scatter(asm_v, [cvec, s, lbase + d], val * m)
                t2 = t + 16
                over = t2 >= L
                t_new = jnp.where(over, t2 - L, t2)
                s_new = s + jnp.where(over, 1, 0)
                return (s_new, t_new)

            lax.fori_loop(0, EPS // 16, ext_body,
                          (jnp.zeros((16,), jnp.int32), lane))

            for c in range(L * D // 128):
                pltpu.sync_copy(
                    asm_v.at[c],
                    out.at[pl.ds(base + k * RPS, RPS),
                           pl.ds(f * L * D + c * 128, 128)])
            return carry

        lax.fori_loop(0, NSUB, sub_body, 0)


_sc_call = functools.partial(
    pl.kernel,
    mesh=plsc.VectorSubcoreMesh(core_axis_name="c", subcore_axis_name="s"),
    compiler_params=pltpu.CompilerParams(needs_layout_passes=False,
                                         use_tc_tiling_on_sc=True),
    out_type=jax.ShapeDtypeStruct((B, 2 * L * D), jnp.float32),
    scratch_types=[
        pltpu.VMEM((B + 128,), jnp.int32),
        pltpu.VMEM((NE,), jnp.int32),
        pltpu.VMEM((NE,), jnp.float32),
        pltpu.VMEM((NE,), jnp.int32),
        pltpu.VMEM((NE,), jnp.int32),
        pltpu.VMEM((EPS, 128), jnp.float32),
        pltpu.VMEM((L * D // 128, RPS, 128), jnp.float32),
        pltpu.SemaphoreType.DMA,
    ],
)(_sc_body)


def kernel(indices_0, offsets_0, indices_1, offsets_1, table_0, table_1):
    tab0r = table_0.reshape(V // 4, 4 * D)
    tab1r = table_1.reshape(V // 4, 4 * D)
    return _sc_call(indices_0, offsets_0, indices_1, offsets_1, tab0r, tab1r)
